# baseline (speedup 1.0000x reference)
import functools
import jax
import jax.numpy as jnp
from jax import lax
from jax.experimental import pallas as pl
from jax.experimental.pallas import tpu as pltpu
from jax.experimental.pallas import tpu_sc as plsc

_NC = 2
_NS = 16
_NW = _NC * _NS
_TR = 32
_B_TC = 2
_MBLK = 512


def _tc_body(x_ref, p_ref, o_ref):
    o_ref[...] = x_ref[...] + p_ref[...]


def _tc_part(x2, pos_table, B_tc, M, D):
    grid = (M // _MBLK, B_tc)
    return pl.pallas_call(
        _tc_body,
        grid=grid,
        in_specs=[
            pl.BlockSpec((_MBLK, D), lambda i, b: (b * (M // _MBLK) + i, 0)),
            pl.BlockSpec((_MBLK, D), lambda i, b: (i, 0)),
        ],
        out_specs=pl.BlockSpec((_MBLK, D), lambda i, b: (b * (M // _MBLK) + i, 0)),
        out_shape=jax.ShapeDtypeStruct((B_tc * M, D), jnp.float32),
    )(x2, pos_table)


def _sc_part(x2, pos_table, b0, B, M, D):
    nb = B - b0
    mw = M // _NW
    nt = mw // _TR
    mesh = plsc.VectorSubcoreMesh(core_axis_name="c", subcore_axis_name="s")

    @functools.partial(
        pl.kernel,
        mesh=mesh,
        out_type=jax.ShapeDtypeStruct((nb * M, D), jnp.float32),
        scratch_types=[
            pltpu.VMEM((_TR, D), jnp.float32),
            pltpu.VMEM((_TR, D), jnp.float32),
            pltpu.VMEM((_TR, D), jnp.float32),
            pltpu.SemaphoreType.DMA,
            pltpu.SemaphoreType.DMA,
            pltpu.SemaphoreType.DMA,
            pltpu.SemaphoreType.DMA,
        ],
    )
    def k(x_hbm, p_hbm, o_hbm, pbuf, ob0, ob1, si0, si1, so0, so1):
        c = lax.axis_index("c")
        s = lax.axis_index("s")
        wid = s * _NC + c
        m0 = wid * mw
        ob = (ob0, ob1)
        si = (si0, si1)
        so = (so0, so1)

        def add_tile(buf):
            def row_body(r, _):
                for j in range(D // 16):
                    sl = pl.ds(j * 16, 16)
                    plsc.addupdate(buf.at[r, sl], pbuf[r, sl])
                return 0

            lax.fori_loop(0, _TR, row_body, 0)

        def tile_body(t, _):
            prow = m0 + t * _TR
            pltpu.sync_copy(p_hbm.at[pl.ds(prow, _TR)], pbuf)
            din = [None] * nb
            dout = [None] * nb
            din[0] = pltpu.async_copy(
                x_hbm.at[pl.ds(b0 * M + prow, _TR)], ob[0], si[0]
            )
            for i in range(nb):
                if i + 1 < nb:
                    if i >= 1:
                        dout[i - 1].wait()
                    nxt = (b0 + i + 1) * M + prow
                    din[i + 1] = pltpu.async_copy(
                        x_hbm.at[pl.ds(nxt, _TR)], ob[(i + 1) % 2], si[(i + 1) % 2]
                    )
                din[i].wait()
                add_tile(ob[i % 2])
                dout[i] = pltpu.async_copy(
                    ob[i % 2], o_hbm.at[pl.ds(i * M + prow, _TR)], so[i % 2]
                )
            if nb >= 2:
                dout[nb - 2].wait()
            dout[nb - 1].wait()
            return 0

        lax.fori_loop(0, nt, tile_body, 0)

    return k(x2, pos_table)


def kernel(x, pos_table):
    B, M, D = x.shape
    x2 = x.reshape(B * M, D)
    sc_out = _sc_part(x2, pos_table, _B_TC, B, M, D)
    tc_out = _tc_part(x2, pos_table, _B_TC, M, D)
    out2 = jnp.concatenate([tc_out, sc_out], axis=0)
    return out2.reshape(B, M, D)


# speedup vs baseline: 2.4692x; 2.4692x over previous
import jax
import jax.numpy as jnp
from jax.experimental import pallas as pl

_MBLK = 512


def _add_body(x_ref, p_ref, o_ref):
    o_ref[...] = x_ref[...] + p_ref[...][None, :, :]


def kernel(x, pos_table):
    B, M, D = x.shape
    grid = (M // _MBLK,)
    return pl.pallas_call(
        _add_body,
        grid=grid,
        in_specs=[
            pl.BlockSpec((B, _MBLK, D), lambda i: (0, i, 0)),
            pl.BlockSpec((_MBLK, D), lambda i: (i, 0)),
        ],
        out_specs=pl.BlockSpec((B, _MBLK, D), lambda i: (0, i, 0)),
        out_shape=jax.ShapeDtypeStruct((B, M, D), x.dtype),
    )(x, pos_table)


# speedup vs baseline: 2.4909x; 1.0088x over previous
import jax
import jax.numpy as jnp
from jax.experimental import pallas as pl

_MBLK = 1024


def _add_body(x_ref, p_ref, o_ref):
    o_ref[...] = x_ref[...] + p_ref[...][None, :, :]


def kernel(x, pos_table):
    B, M, D = x.shape
    grid = (M // _MBLK,)
    return pl.pallas_call(
        _add_body,
        grid=grid,
        in_specs=[
            pl.BlockSpec((B, _MBLK, D), lambda i: (0, i, 0)),
            pl.BlockSpec((_MBLK, D), lambda i: (i, 0)),
        ],
        out_specs=pl.BlockSpec((B, _MBLK, D), lambda i: (0, i, 0)),
        out_shape=jax.ShapeDtypeStruct((B, M, D), x.dtype),
    )(x, pos_table)
